# baseline (speedup 1.0000x reference)
import jax
import jax.numpy as jnp
from jax.experimental import pallas as pl
from jax.experimental.pallas import tpu as pltpu

_TM = 200


def _gcn_block(at_ref, ab_ref, x_ref, w_ref, o_ref):
    wt = w_ref[...].T
    axt = jnp.dot(at_ref[...], x_ref[...], preferred_element_type=jnp.float32)
    o_ref[0] = jnp.dot(axt, wt, preferred_element_type=jnp.float32)
    axb = jnp.dot(ab_ref[...], x_ref[...], preferred_element_type=jnp.float32)
    o_ref[1] = jnp.dot(axb, wt, preferred_element_type=jnp.float32)


def kernel(X, A_tilde, W):
    n, d_in = X.shape
    d_out = W.shape[0]
    nblk = (n // 2) // _TM
    out = pl.pallas_call(
        _gcn_block,
        grid=(nblk,),
        in_specs=[
            pl.BlockSpec((_TM, n), lambda i: (i, 0)),
            pl.BlockSpec((_TM, n), lambda i, _nblk=nblk: (i + _nblk, 0)),
            pl.BlockSpec((n, d_in), lambda i: (0, 0)),
            pl.BlockSpec((d_out, d_in), lambda i: (0, 0)),
        ],
        out_specs=pl.BlockSpec((2, _TM, d_out), lambda i: (0, i, 0)),
        out_shape=jax.ShapeDtypeStruct((2, n // 2, d_out), jnp.float32),
        compiler_params=pltpu.CompilerParams(dimension_semantics=("parallel",)),
    )(A_tilde, A_tilde, X, W)
    return out.reshape(n, d_out)


# speedup vs baseline: 1.0946x; 1.0946x over previous
import jax
import jax.numpy as jnp
from jax.experimental import pallas as pl
from jax.experimental.pallas import tpu as pltpu

_TM = 400


def _gcn_block(a_ref, x_ref, w_ref, o_ref):
    ax = jnp.dot(a_ref[...], x_ref[...], preferred_element_type=jnp.float32)
    o_ref[...] = jnp.dot(ax, w_ref[...].T, preferred_element_type=jnp.float32)


def kernel(X, A_tilde, W):
    n, d_in = X.shape
    d_out = W.shape[0]
    return pl.pallas_call(
        _gcn_block,
        grid=(n // _TM,),
        in_specs=[
            pl.BlockSpec((_TM, n), lambda i: (i, 0)),
            pl.BlockSpec((n, d_in), lambda i: (0, 0)),
            pl.BlockSpec((d_out, d_in), lambda i: (0, 0)),
        ],
        out_specs=pl.BlockSpec((_TM, d_out), lambda i: (i, 0)),
        out_shape=jax.ShapeDtypeStruct((n, d_out), jnp.float32),
        compiler_params=pltpu.CompilerParams(dimension_semantics=("parallel",)),
    )(A_tilde, X, W)
